# baseline (speedup 1.0000x reference)
import functools

import jax
import jax.numpy as jnp
from jax import lax
from jax.experimental import pallas as pl
from jax.experimental.pallas import tpu as pltpu
from jax.experimental.pallas import tpu_sc as plsc

N_USERS = 1000000
N_ITEMS = 1000000
EMBED_DIM = 64
BATCH = 16384

NC, NS, L = 2, 16, 16
NW = NC * NS
B_PER_W = BATCH // NW
GROUPS = B_PER_W // L
CHUNKS = EMBED_DIM // L


def _body(user_hbm, item_hbm, uid_hbm, iid_hbm, out_hbm,
          uidx_v, iidx_v, urows_v, irows_v, out_v, tr_v, sem_u, sem_i):
    wid = lax.axis_index("s") * NC + lax.axis_index("c")
    base = wid * B_PER_W

    pltpu.sync_copy(uid_hbm.at[pl.ds(base, B_PER_W)], uidx_v)
    pltpu.sync_copy(iid_hbm.at[pl.ds(base, B_PER_W)], iidx_v)
    cp_u = pltpu.async_copy(user_hbm.at[uidx_v], urows_v, sem_u)
    cp_i = pltpu.async_copy(item_hbm.at[iidx_v], irows_v, sem_i)
    cp_u.wait()
    cp_i.wait()

    lane = lax.iota(jnp.int32, L)
    col_addr = lane * (L + 1)

    def group(g, carry):
        row0 = g * L
        for r in range(L):
            s = urows_v[row0 + r, pl.ds(0, L)] * irows_v[row0 + r, pl.ds(0, L)]
            for c in range(1, CHUNKS):
                s = s + (urows_v[row0 + r, pl.ds(c * L, L)]
                         * irows_v[row0 + r, pl.ds(c * L, L)])
            plsc.store_scatter(tr_v, [col_addr + r], s)
        acc = tr_v[pl.ds(0, L)]
        for l in range(1, L):
            acc = acc + tr_v[pl.ds(l * (L + 1), L)]
        out_v[pl.ds(row0, L)] = acc
        return carry

    lax.fori_loop(0, GROUPS, group, 0)

    pltpu.sync_copy(out_v, out_hbm.at[pl.ds(base, B_PER_W)])


@functools.partial(
    pl.kernel,
    out_type=jax.ShapeDtypeStruct((BATCH,), jnp.float32),
    mesh=plsc.VectorSubcoreMesh(core_axis_name="c", subcore_axis_name="s",
                                num_cores=NC, num_subcores=NS),
    compiler_params=pltpu.CompilerParams(needs_layout_passes=False,
                                         use_tc_tiling_on_sc=False),
    scratch_types=[
        pltpu.VMEM((B_PER_W,), jnp.int32),
        pltpu.VMEM((B_PER_W,), jnp.int32),
        pltpu.VMEM((B_PER_W, EMBED_DIM), jnp.float32),
        pltpu.VMEM((B_PER_W, EMBED_DIM), jnp.float32),
        pltpu.VMEM((B_PER_W,), jnp.float32),
        pltpu.VMEM((L * (L + 1),), jnp.float32),
        pltpu.SemaphoreType.DMA,
        pltpu.SemaphoreType.DMA,
    ],
)
def _mf_bpr_sc(*args):
    _body(*args)


def kernel(uid, iid, user_table, item_table):
    return _mf_bpr_sc(user_table, item_table, uid, iid)
